# baseline (speedup 1.0000x reference)
import jax
import jax.numpy as jnp
from jax import lax
from jax.experimental import pallas as pl

T = 16384
D = 128
H = 256
E = 3
B = 2048


def _moe_body(x_ref, wg_ref, w1_ref, b1_ref, w2_ref, b2_ref, out_ref):
    x = x_ref[...]
    logits = jnp.dot(x, wg_ref[...],
                     preferred_element_type=jnp.float32)
    lm = jnp.max(logits, axis=-1, keepdims=True)
    top_v = 1.0 / jnp.sum(jnp.exp(logits - lm), axis=-1)
    top_i = jnp.argmax(logits, axis=-1)

    hs = [jnp.dot(x, w1_ref[e], preferred_element_type=jnp.float32)
          for e in range(E)]
    ti = top_i[:, None]
    hsel = jnp.where(ti == 0, hs[0],
                     jnp.where(ti == 1, hs[1], hs[2]))
    b1sel = jnp.where(ti == 0, b1_ref[0][None, :],
                      jnp.where(ti == 1, b1_ref[1][None, :],
                                b1_ref[2][None, :]))
    hb = hsel + b1sel
    c1 = 0.7978845608028654
    c2 = 0.044715 * c1
    u = hb * (c1 + c2 * (hb * hb))
    a = (0.5 * top_v)[:, None] * hb
    th = jnp.tanh(u)
    gh = a + a * th
    ys = [jnp.dot(gh, w2_ref[e], preferred_element_type=jnp.float32)
          for e in range(E)]
    y = jnp.where(ti == 0, ys[0], jnp.where(ti == 1, ys[1], ys[2]))
    b2sel = jnp.where(ti == 0, b2_ref[0][None, :],
                      jnp.where(ti == 1, b2_ref[1][None, :],
                                b2_ref[2][None, :]))
    out_ref[...] = y + top_v[:, None] * b2sel


@jax.jit
def kernel(x, Wg, W1, b1, W2, b2):
    return pl.pallas_call(
        _moe_body,
        grid=(T // B,),
        in_specs=[
            pl.BlockSpec((B, D), lambda i: (i, 0)),
            pl.BlockSpec((D, E), lambda i: (0, 0)),
            pl.BlockSpec((E, D, H), lambda i: (0, 0, 0)),
            pl.BlockSpec((E, H), lambda i: (0, 0)),
            pl.BlockSpec((E, H, D), lambda i: (0, 0, 0)),
            pl.BlockSpec((E, D), lambda i: (0, 0)),
        ],
        out_specs=pl.BlockSpec((B, D), lambda i: (i, 0)),
        out_shape=jax.ShapeDtypeStruct((T, D), jnp.float32),
    )(x, Wg, W1, b1, W2, b2)


# speedup vs baseline: 1.0326x; 1.0326x over previous
import jax
import jax.numpy as jnp
from jax.experimental import pallas as pl

T = 16384
D = 128
H = 256
E = 3
B = 4096


def _moe_body(x_ref, wg_ref, w1_ref, b1_ref, w2_ref, b2_ref, out_ref):
    x = x_ref[...]
    logits = jnp.dot(x, wg_ref[...],
                     preferred_element_type=jnp.float32)
    lm = jnp.max(logits, axis=-1, keepdims=True)
    top_v = 1.0 / jnp.sum(jnp.exp(logits - lm), axis=-1)
    top_i = jnp.argmax(logits, axis=-1)

    hs = [jnp.dot(x, w1_ref[e], preferred_element_type=jnp.float32)
          for e in range(E)]
    ti = top_i[:, None]
    hsel = jnp.where(ti == 0, hs[0],
                     jnp.where(ti == 1, hs[1], hs[2]))
    b1sel = jnp.where(ti == 0, b1_ref[0][None, :],
                      jnp.where(ti == 1, b1_ref[1][None, :],
                                b1_ref[2][None, :]))
    hb = hsel + b1sel
    c1 = 0.7978845608028654
    c2 = 0.044715 * c1
    u = hb * (c1 + c2 * (hb * hb))
    a = (0.5 * top_v)[:, None] * hb
    th = jnp.tanh(u)
    gh = a + a * th
    ys = [jnp.dot(gh, w2_ref[e], preferred_element_type=jnp.float32)
          for e in range(E)]
    y = jnp.where(ti == 0, ys[0], jnp.where(ti == 1, ys[1], ys[2]))
    b2sel = jnp.where(ti == 0, b2_ref[0][None, :],
                      jnp.where(ti == 1, b2_ref[1][None, :],
                                b2_ref[2][None, :]))
    out_ref[...] = y + top_v[:, None] * b2sel


@jax.jit
def kernel(x, Wg, W1, b1, W2, b2):
    return pl.pallas_call(
        _moe_body,
        grid=(T // B,),
        in_specs=[
            pl.BlockSpec((B, D), lambda i: (i, 0)),
            pl.BlockSpec((D, E), lambda i: (0, 0)),
            pl.BlockSpec((E, D, H), lambda i: (0, 0, 0)),
            pl.BlockSpec((E, H), lambda i: (0, 0)),
            pl.BlockSpec((E, H, D), lambda i: (0, 0, 0)),
            pl.BlockSpec((E, D), lambda i: (0, 0)),
        ],
        out_specs=pl.BlockSpec((B, D), lambda i: (i, 0)),
        out_shape=jax.ShapeDtypeStruct((T, D), jnp.float32),
    )(x, Wg, W1, b1, W2, b2)
